# Initial kernel scaffold; baseline (speedup 1.0000x reference)
import jax
import jax.numpy as jnp
from jax.experimental import pallas as pl


def kernel(tokens, gate_w, w1, b1, w2, b2):
    raise NotImplementedError("write your pallas kernel here")


# speedup vs baseline: 5.7306x; 5.7306x over previous
import jax
import jax.numpy as jnp
from jax import lax
from jax.experimental import pallas as pl
from jax.experimental.pallas import tpu as pltpu

_BT = 512


def _gate_kernel(x_ref, gw_ref, wtok_ref, cnt_ref):
    x = x_ref[:]
    gw = gw_ref[:]
    logits = lax.dot_general(x, gw, (((1,), (1,)), ((), ())),
                             preferred_element_type=jnp.float32)
    m = jnp.max(logits, axis=1, keepdims=True)
    z = jnp.exp(logits - m)
    p = z / jnp.sum(z, axis=1, keepdims=True)
    e_count = p.shape[1]
    eio = lax.broadcasted_iota(jnp.int32, p.shape, 1)
    m1 = jnp.max(p, axis=1, keepdims=True)
    i1 = jnp.min(jnp.where(p == m1, eio, e_count), axis=1, keepdims=True)
    p2m = jnp.where(eio == i1, -1.0, p)
    m2 = jnp.max(p2m, axis=1, keepdims=True)
    i2 = jnp.min(jnp.where(p2m == m2, eio, e_count), axis=1, keepdims=True)
    sel = (eio == i1) | (eio == i2)
    wtok_ref[:] = jnp.where(sel, p, 0.0)
    partial = jnp.sum(sel.astype(jnp.int32), axis=0, keepdims=True)

    @pl.when(pl.program_id(0) == 0)
    def _():
        cnt_ref[:] = partial

    @pl.when(pl.program_id(0) != 0)
    def _():
        cnt_ref[:] += partial


def _ffn_kernel(flags_ref, amap_ref, x_ref, wt_ref, w1_ref, b1_ref,
                w2_ref, b2_ref, out_ref):
    del amap_ref
    e = pl.program_id(1)

    @pl.when(e == 0)
    def _():
        out_ref[:] = jnp.zeros_like(out_ref)

    @pl.when(flags_ref[e] != 0)
    def _():
        x = x_ref[:]
        h = lax.dot_general(x, w1_ref[0], (((1,), (1,)), ((), ())),
                            preferred_element_type=jnp.float32)
        h = jnp.maximum(h + b1_ref[0], 0.0)
        oe = lax.dot_general(h, w2_ref[0], (((1,), (1,)), ((), ())),
                             preferred_element_type=jnp.float32) + b2_ref[0]
        wt = wt_ref[:]
        lane = lax.broadcasted_iota(jnp.int32, wt.shape, 1)
        wcol = jnp.sum(jnp.where(lane == e, wt, 0.0), axis=1, keepdims=True)
        out_ref[:] += oe * wcol


def kernel(tokens, gate_w, w1, b1, w2, b2):
    batch, seq, hidden = tokens.shape
    n_tok = batch * seq
    n_exp = gate_w.shape[0]
    x = tokens.reshape(n_tok, hidden)
    cap = int(1.25 * n_tok / n_exp)
    nb = n_tok // _BT

    wtok, counts = pl.pallas_call(
        _gate_kernel,
        grid=(nb,),
        in_specs=[
            pl.BlockSpec((_BT, hidden), lambda i: (i, 0)),
            pl.BlockSpec((n_exp, hidden), lambda i: (0, 0)),
        ],
        out_specs=[
            pl.BlockSpec((_BT, n_exp), lambda i: (i, 0)),
            pl.BlockSpec((1, n_exp), lambda i: (0, 0)),
        ],
        out_shape=[
            jax.ShapeDtypeStruct((n_tok, n_exp), jnp.float32),
            jax.ShapeDtypeStruct((1, n_exp), jnp.int32),
        ],
    )(x, gate_w)

    counts = counts[0]
    active = ((counts > 0) & (counts <= cap)).astype(jnp.int32)
    eids = jnp.arange(n_exp, dtype=jnp.int32)
    amap = lax.cummax(jnp.where(active == 1, eids, 0))

    grid_spec = pltpu.PrefetchScalarGridSpec(
        num_scalar_prefetch=2,
        grid=(nb, n_exp),
        in_specs=[
            pl.BlockSpec((_BT, hidden), lambda i, e, f, a: (i, 0)),
            pl.BlockSpec((_BT, n_exp), lambda i, e, f, a: (i, 0)),
            pl.BlockSpec((1, hidden, hidden), lambda i, e, f, a: (a[e], 0, 0)),
            pl.BlockSpec((1, 1, hidden), lambda i, e, f, a: (a[e], 0, 0)),
            pl.BlockSpec((1, hidden, hidden), lambda i, e, f, a: (a[e], 0, 0)),
            pl.BlockSpec((1, 1, hidden), lambda i, e, f, a: (a[e], 0, 0)),
        ],
        out_specs=pl.BlockSpec((_BT, hidden), lambda i, e, f, a: (i, 0)),
    )
    out = pl.pallas_call(
        _ffn_kernel,
        grid_spec=grid_spec,
        out_shape=jax.ShapeDtypeStruct((n_tok, hidden), jnp.float32),
        compiler_params=pltpu.CompilerParams(
            dimension_semantics=("arbitrary", "arbitrary")),
    )(active, amap, x, wtok, w1, b1.reshape(n_exp, 1, hidden),
      w2, b2.reshape(n_exp, 1, hidden))

    return out.reshape(batch, seq, hidden)
